# baseline (speedup 1.0000x reference)
import functools

import jax
import jax.numpy as jnp
from jax.experimental import pallas as pl
from jax.experimental.pallas import tpu as pltpu

_EPS = 1e-05


def _fused_kernel(x1_ref, x2_ref, w_ref, b_ref, gamma_ref, beta_ref,
                  out_add_ref, mean_ref, rstd_ref, out_ref, wbf_ref, *, n_b):
    i = pl.program_id(0)

    @pl.when(i == 0)
    def _():
        wbf_ref[...] = w_ref[...].astype(jnp.bfloat16)

    x = x1_ref[...] + x2_ref[...]
    out_add_ref[...] = x
    mean = jnp.mean(x, axis=1, keepdims=True)
    xc = x - mean
    var = jnp.mean(xc * xc, axis=1, keepdims=True)
    rstd = jax.lax.rsqrt(var + _EPS)
    brow = i % n_b
    mean_ref[pl.ds(brow, 1), :] = mean.reshape(1, -1)
    rstd_ref[pl.ds(brow, 1), :] = rstd.reshape(1, -1)
    ln = ((xc * rstd) * gamma_ref[...] + beta_ref[...]).astype(jnp.bfloat16)
    out_ref[...] = (
        jnp.dot(ln, wbf_ref[...], preferred_element_type=jnp.float32)
        + b_ref[...]
    )


def kernel(x1, x2, w, b, gamma, beta):
    B, M, N = x1.shape
    D = w.shape[1]
    R = B * M
    TM = 512
    n_m = M // TM
    n_b = B

    x1f = x1.reshape(R, N)
    x2f = x2.reshape(R, N)
    b2 = b.reshape(1, D)
    gamma2 = gamma.reshape(1, N)
    beta2 = beta.reshape(1, N)

    def row_tile(i):
        return (i % n_b) * n_m + i // n_b

    body = functools.partial(_fused_kernel, n_b=n_b)

    out_add, mean, rstd, out = pl.pallas_call(
        body,
        grid=(R // TM,),
        in_specs=[
            pl.BlockSpec((TM, N), lambda i: (row_tile(i), 0)),
            pl.BlockSpec((TM, N), lambda i: (row_tile(i), 0)),
            pl.BlockSpec((N, D), lambda i: (0, 0)),
            pl.BlockSpec((1, D), lambda i: (0, 0)),
            pl.BlockSpec((1, N), lambda i: (0, 0)),
            pl.BlockSpec((1, N), lambda i: (0, 0)),
        ],
        out_specs=[
            pl.BlockSpec((TM, N), lambda i: (row_tile(i), 0)),
            pl.BlockSpec((B, TM), lambda i: (0, i // n_b)),
            pl.BlockSpec((B, TM), lambda i: (0, i // n_b)),
            pl.BlockSpec((TM, D), lambda i: (row_tile(i), 0)),
        ],
        out_shape=[
            jax.ShapeDtypeStruct((R, N), jnp.float32),
            jax.ShapeDtypeStruct((B, M), jnp.float32),
            jax.ShapeDtypeStruct((B, M), jnp.float32),
            jax.ShapeDtypeStruct((R, D), jnp.float32),
        ],
        scratch_shapes=[pltpu.VMEM((N, D), jnp.bfloat16)],
        compiler_params=pltpu.CompilerParams(
            dimension_semantics=("parallel",),
            vmem_limit_bytes=56 * 1024 * 1024,
        ),
        name="addln_matmul_fused",
    )(x1f, x2f, w, b2, gamma2, beta2)

    return (
        out_add.reshape(B, M, N),
        mean,
        rstd,
        out.reshape(B, M, D),
    )
